# Initial kernel scaffold; baseline (speedup 1.0000x reference)
import jax
import jax.numpy as jnp
from jax.experimental import pallas as pl


def kernel(x, edge_index, edge_attr, batch, params):
    raise NotImplementedError("write your pallas kernel here")


# speedup vs baseline: 2.9448x; 2.9448x over previous
import functools

import jax
import jax.numpy as jnp
from jax import lax
from jax.experimental import pallas as pl
from jax.experimental.pallas import tpu as pltpu
from jax.experimental.pallas import tpu_sc as plsc

N = 10000
E = 320000
D = 128
DE = 16
H = 128
NL = 3
G = 64

_NC = 2
_NS = 16
_NW = _NC * _NS
_EW = E // _NW
_C = 80
_NCHUNK = _EW // _C
_RC = 80
_NRC = N // _RC

_f32 = jnp.float32


_EB = 4000


def _edge_body(ea_ref, w_ref, b_ref, e0_ref, e1_ref, e2_ref):
    r = jnp.dot(ea_ref[...].astype(jnp.bfloat16), w_ref[...].astype(jnp.bfloat16),
                preferred_element_type=_f32) + b_ref[...]
    e0_ref[...] = r[:, 0:H]
    e1_ref[...] = r[:, H:2 * H]
    e2_ref[...] = r[:, 2 * H:3 * H]


def _edge_feats(ea, wcat, bcat):
    eshape = jax.ShapeDtypeStruct((E, H), _f32)
    return pl.pallas_call(
        _edge_body,
        grid=(E // _EB,),
        in_specs=[
            pl.BlockSpec((_EB, DE), lambda i: (i, 0)),
            pl.BlockSpec((DE, 3 * H), lambda i: (0, 0)),
            pl.BlockSpec((1, 3 * H), lambda i: (0, 0)),
        ],
        out_specs=[
            pl.BlockSpec((_EB, H), lambda i: (i, 0)),
            pl.BlockSpec((_EB, H), lambda i: (i, 0)),
            pl.BlockSpec((_EB, H), lambda i: (i, 0)),
        ],
        out_shape=[eshape, eshape, eshape],
    )(ea, wcat, bcat)


def _sc_body(h_hbm, e_hbm, src_hbm, dst_hbm, out_hbm,
             src_v, dst_v, rows_v, e_v, zbuf, agg_sh, sem):
    c = lax.axis_index("c")
    s = lax.axis_index("s")
    wid = s * _NC + c

    def _zrow(i, _):
        for j in range(H // 16):
            zbuf[i, pl.ds(j * 16, 16)] = jnp.zeros((16,), _f32)
        return 0
    lax.fori_loop(0, _RC, _zrow, 0)
    nrc = (_NRC - s + _NS - 1) // _NS

    def _zchunk(t, _):
        r0 = (t * _NS + s) * _RC
        pltpu.sync_copy(zbuf, agg_sh.at[pl.ds(r0, _RC), :])
        return 0
    lax.fori_loop(0, nrc, _zchunk, 0)
    plsc.subcore_barrier()

    def _chunk(t, _):
        base = wid * _EW + t * _C
        pltpu.sync_copy(src_hbm.at[pl.ds(base, _C)], src_v)
        pltpu.sync_copy(dst_hbm.at[pl.ds(base, _C)], dst_v)
        cp = pltpu.async_copy(h_hbm.at[src_v], rows_v, sem)
        pltpu.sync_copy(e_hbm.at[pl.ds(base, _C), :], e_v)
        cp.wait()

        def _row(i, _):
            for j in range(H // 16):
                sl = pl.ds(j * 16, 16)
                rows_v[i, sl] = jnp.maximum(rows_v[i, sl] + e_v[i, sl], 0.0)
            return 0
        lax.fori_loop(0, _C, _row, 0)
        pltpu.sync_copy(rows_v, agg_sh.at[dst_v], add=True)
        return 0
    lax.fori_loop(0, _NCHUNK, _chunk, 0)
    plsc.subcore_barrier()

    def _wchunk(t, _):
        r0 = (t * _NS + s) * _RC
        pltpu.sync_copy(agg_sh.at[pl.ds(r0, _RC), :], zbuf)
        pltpu.sync_copy(zbuf, out_hbm.at[c, pl.ds(r0, _RC), :])
        return 0
    lax.fori_loop(0, nrc, _wchunk, 0)


_sc_aggregate = functools.partial(
    pl.kernel,
    out_type=jax.ShapeDtypeStruct((_NC, N, H), _f32),
    mesh=plsc.VectorSubcoreMesh(core_axis_name="c", subcore_axis_name="s"),
    scratch_types=[
        pltpu.VMEM((_C,), jnp.int32),
        pltpu.VMEM((_C,), jnp.int32),
        pltpu.VMEM((_C, H), _f32),
        pltpu.VMEM((_C, H), _f32),
        pltpu.VMEM((_RC, H), _f32),
        pltpu.VMEM_SHARED((N, H), _f32),
        pltpu.SemaphoreType.DMA,
    ],
)(_sc_body)


_NB = 1000


def _update_body(h_ref, agg_ref, w1_ref, b1_ref, w2_ref, b2_ref, s_ref, o_ref):
    a = agg_ref[0] + agg_ref[1]
    z = h_ref[...] * s_ref[0, 0] + a
    z = jnp.maximum(jnp.dot(z.astype(jnp.bfloat16), w1_ref[...].astype(jnp.bfloat16),
                            preferred_element_type=_f32) + b1_ref[...], 0.0)
    o_ref[...] = jnp.dot(z.astype(jnp.bfloat16), w2_ref[...].astype(jnp.bfloat16),
                         preferred_element_type=_f32) + b2_ref[...]


def _node_update(h, agg2, w1, b1, w2, b2, scale):
    return pl.pallas_call(
        _update_body,
        grid=(N // _NB,),
        in_specs=[
            pl.BlockSpec((_NB, H), lambda i: (i, 0)),
            pl.BlockSpec((_NC, _NB, H), lambda i: (0, i, 0)),
            pl.BlockSpec((H, H), lambda i: (0, 0)),
            pl.BlockSpec((1, H), lambda i: (0, 0)),
            pl.BlockSpec((H, H), lambda i: (0, 0)),
            pl.BlockSpec((1, H), lambda i: (0, 0)),
            pl.BlockSpec((1, 1), lambda i: (0, 0)),
        ],
        out_specs=pl.BlockSpec((_NB, H), lambda i: (i, 0)),
        out_shape=jax.ShapeDtypeStruct((N, H), _f32),
    )(h, agg2, w1, b1, w2, b2, scale)


def _final_body(h1_ref, h2_ref, h3_ref, b3_ref, w0_ref, w1_ref, w2_ref,
                bjk_ref, bns_ref, bnb_ref, bng_ref, bnbeta_ref,
                wh1_ref, bh1_ref, wh2_ref, bh2_ref,
                o_ref, sums, cnt):
    i = pl.program_id(0)

    @pl.when(i == 0)
    def _():
        sums[...] = jnp.zeros_like(sums)
        cnt[...] = jnp.zeros_like(cnt)

    hjk = (jnp.dot(h1_ref[...].astype(jnp.bfloat16), w0_ref[...].astype(jnp.bfloat16),
                   preferred_element_type=_f32)
           + jnp.dot(h2_ref[...].astype(jnp.bfloat16), w1_ref[...].astype(jnp.bfloat16),
                     preferred_element_type=_f32)
           + jnp.dot(h3_ref[...].astype(jnp.bfloat16), w2_ref[...].astype(jnp.bfloat16),
                     preferred_element_type=_f32)
           + bjk_ref[...])
    b = b3_ref[0, 0, :]
    oh = (b[:, None] == lax.broadcasted_iota(jnp.int32, (_NB, G), 1)).astype(_f32)
    sums[...] += lax.dot_general(oh, hjk, (((0,), (0,)), ((), ())),
                                 preferred_element_type=_f32, precision=lax.Precision.HIGHEST)
    cnt[...] += lax.dot_general(oh, jnp.ones((_NB, 1), _f32),
                                (((0,), (0,)), ((), ())),
                                preferred_element_type=_f32, precision=lax.Precision.DEFAULT)

    @pl.when(i == N // _NB - 1)
    def _():
        pooled = sums[...] / jnp.maximum(cnt[...], 1.0)
        q = ((pooled - bns_ref[...]) / jnp.sqrt(bnb_ref[...] + 1e-5)
             * bng_ref[...] + bnbeta_ref[...])
        q = jnp.maximum(jnp.dot(q.astype(jnp.bfloat16), wh1_ref[...].astype(jnp.bfloat16),
                                preferred_element_type=_f32) + bh1_ref[...], 0.0)
        o_ref[...] = (jnp.dot(q.astype(jnp.bfloat16), wh2_ref[...].astype(jnp.bfloat16),
                              preferred_element_type=_f32) + bh2_ref[...])


def _final(h1, h2, h3, batch3, wjk, bjk, bnm, bnv, bng, bnbeta,
           wh1, bh1, wh2, bh2):
    full = lambda shape: pl.BlockSpec(shape, lambda i: tuple(0 for _ in shape))
    return pl.pallas_call(
        _final_body,
        grid=(N // _NB,),
        in_specs=[
            pl.BlockSpec((_NB, H), lambda i: (i, 0)),
            pl.BlockSpec((_NB, H), lambda i: (i, 0)),
            pl.BlockSpec((_NB, H), lambda i: (i, 0)),
            pl.BlockSpec((1, 1, _NB), lambda i: (i, 0, 0)),
            full((H, H)), full((H, H)), full((H, H)),
            full((1, H)), full((1, H)), full((1, H)), full((1, H)), full((1, H)),
            full((H, G)), full((1, G)), full((G, 1)), full((1, 1)),
        ],
        out_specs=full((G, 1)),
        out_shape=jax.ShapeDtypeStruct((G, 1), _f32),
        scratch_shapes=[
            pltpu.VMEM((G, H), _f32),
            pltpu.VMEM((G, 1), _f32),
        ],
    )(h1, h2, h3, batch3, wjk[0], wjk[1], wjk[2], bjk, bnm, bnv, bng, bnbeta,
      wh1, bh1, wh2, bh2)


def kernel(x, edge_index, edge_attr, batch, params):
    src = edge_index[0].astype(jnp.int32)
    dst = edge_index[1].astype(jnp.int32)
    h = x.astype(_f32)
    ea = edge_attr.astype(_f32)

    wcat = jnp.concatenate([params["conv%d" % l]["We"] for l in range(NL)], axis=1)
    bcat = jnp.concatenate([params["conv%d" % l]["be"] for l in range(NL)]).reshape(1, 3 * H)
    es = _edge_feats(ea, wcat, bcat)

    outs = []
    for l in range(NL):
        p = params["conv%d" % l]
        agg2 = _sc_aggregate(h, es[l], src, dst)
        scale = (1.0 + p["eps"]).astype(_f32).reshape(1, 1)
        h = _node_update(h, agg2, p["W1"], p["b1"].reshape(1, H),
                         p["W2"], p["b2"].reshape(1, H), scale)
        outs.append(h)

    batch3 = batch.astype(jnp.int32).reshape(N // _NB, 1, _NB)
    wjk = params["Wjk"].reshape(NL, H, H)
    bjk = params["bjk"].reshape(1, H)
    out = _final(outs[0], outs[1], outs[2], batch3, wjk, bjk,
                 params["bn_mean"].reshape(1, H), params["bn_var"].reshape(1, H),
                 params["bn_gamma"].reshape(1, H), params["bn_beta"].reshape(1, H),
                 params["Wh1"], params["bh1"].reshape(1, G),
                 params["Wh2"], params["bh2"].reshape(1, 1))
    return out[:, 0]
